# Initial kernel scaffold; baseline (speedup 1.0000x reference)
import jax
import jax.numpy as jnp
from jax.experimental import pallas as pl


def kernel(features, idx, first_order_neighs, second_order_neighs, W1, b1, W2, b2):
    raise NotImplementedError("write your pallas kernel here")


# speedup vs baseline: 7.2106x; 7.2106x over previous
import functools

import jax
import jax.numpy as jnp
from jax import lax
from jax.experimental import pallas as pl
from jax.experimental.pallas import tpu as pltpu
from jax.experimental.pallas import tpu_sc as plsc

N_NODES = 100000
IN_DIM = 60
BATCH = 16384
FANOUT = 6
DIMS = 20
TD = 24
HD = 128

NC, NS = 2, 16
NW = NC * NS
BPW = BATCH // NW
CH = 64
NCHUNK = BPW // CH
IDX_TILE = 128


def _table_body(x_ref, w1a_ref, w1b_ref, f1_ref, f2_ref):
    x = x_ref[...]
    f1_ref[...] = jnp.dot(x, w1a_ref[...], preferred_element_type=jnp.float32)
    f2_ref[...] = jnp.dot(x, w1b_ref[...], preferred_element_type=jnp.float32)


def _make_tables(features, W1):
    rows = features.shape[0]
    blk = 2048
    grid = (rows + blk - 1) // blk
    return pl.pallas_call(
        _table_body,
        grid=(grid,),
        in_specs=[
            pl.BlockSpec((blk, IN_DIM), lambda i: (i, 0)),
            pl.BlockSpec((IN_DIM, TD), lambda i: (0, 0)),
            pl.BlockSpec((IN_DIM, TD), lambda i: (0, 0)),
        ],
        out_specs=[
            pl.BlockSpec((blk, TD), lambda i: (i, 0)),
            pl.BlockSpec((blk, TD), lambda i: (i, 0)),
        ],
        out_shape=[
            jax.ShapeDtypeStruct((rows, TD), jnp.float32),
            jax.ShapeDtypeStruct((rows, TD), jnp.float32),
        ],
    )(features,
      jnp.pad(W1[:IN_DIM], ((0, 0), (0, TD - DIMS))),
      jnp.pad(W1[IN_DIM:], ((0, 0), (0, TD - DIMS))))


def _head_body(h_ref, w2_ref, b2_ref, o_ref):
    h = h_ref[...][:, :2 * DIMS]
    acc = jnp.dot(h, w2_ref[...], preferred_element_type=jnp.float32)
    o_ref[...] = jnp.maximum(acc + b2_ref[...], 0.0)


def _head(H, W2, b2):
    blk = 2048
    return pl.pallas_call(
        _head_body,
        grid=(BATCH // blk,),
        in_specs=[
            pl.BlockSpec((blk, HD), lambda i: (i, 0)),

            pl.BlockSpec((2 * DIMS, DIMS), lambda i: (0, 0)),
            pl.BlockSpec((1, DIMS), lambda i: (0, 0)),
        ],
        out_specs=pl.BlockSpec((blk, DIMS), lambda i: (i, 0)),
        out_shape=jax.ShapeDtypeStruct((BATCH, DIMS), jnp.float32),
    )(H, W2, b2.reshape(1, DIMS))


def _gather_body(f1_hbm, f2_hbm, idx_hbm, n1_hbm, n2_hbm, b1_hbm, out_hbm,
                 idxv, n1v, n2v, rs, rn1a, rn1b, rn2, hb, b1v, sem):
    wid = lax.axis_index("s") * NC + lax.axis_index("c")
    base = wid * BPW
    pltpu.sync_copy(b1_hbm, b1v)
    b1A = b1v[pl.ds(0, 16)]
    b1B = b1v[pl.ds(16, 16)]
    sixth = jnp.float32(1.0 / FANOUT)

    @pl.loop(0, NCHUNK)
    def chunk(ci):
        cb = base + ci * CH
        pltpu.sync_copy(idx_hbm.at[pl.ds(cb, CH)], idxv)
        pltpu.sync_copy(n1_hbm.at[pl.ds(cb * FANOUT, CH * FANOUT)], n1v)
        pltpu.sync_copy(n2_hbm.at[pl.ds(cb * FANOUT * FANOUT,
                                        CH * FANOUT * FANOUT)], n2v)
        cps = [pltpu.async_copy(f1_hbm.at[idxv], rs, sem)]
        for k in range(CH * FANOUT // IDX_TILE):
            src = pl.ds(k * IDX_TILE, IDX_TILE)
            dst = pl.ds(k * IDX_TILE, IDX_TILE)
            cps.append(pltpu.async_copy(f1_hbm.at[n1v.at[src]], rn1a.at[dst], sem))
            cps.append(pltpu.async_copy(f2_hbm.at[n1v.at[src]], rn1b.at[dst], sem))
        for k in range(CH * FANOUT * FANOUT // IDX_TILE):
            src = pl.ds(k * IDX_TILE, IDX_TILE)
            dst = pl.ds(k * IDX_TILE, IDX_TILE)
            cps.append(pltpu.async_copy(f2_hbm.at[n2v.at[src]], rn2.at[dst], sem))
        for cp in cps:
            cp.wait()

        @pl.loop(0, CH)
        def elem(e):
            zero = jnp.zeros((16,), jnp.float32)
            acc0 = zero
            acc1 = zero
            sb0 = zero
            sb1 = zero
            for i in range(FANOUT):
                g = e * FANOUT + i
                s0 = zero
                s1 = zero
                for j in range(FANOUT):
                    r = g * FANOUT + j
                    s0 = s0 + rn2[r, pl.ds(0, 16)]
                    s1 = s1 + rn2[r, pl.ds(4, 16)]
                q0 = jnp.maximum(rn1a[g, pl.ds(0, 16)] + sixth * s0 + b1A, 0.0)
                q1 = jnp.maximum(rn1a[g, pl.ds(4, 16)] + sixth * s1 + b1B, 0.0)
                acc0 = acc0 + q0
                acc1 = acc1 + q1
                sb0 = sb0 + rn1b[g, pl.ds(0, 16)]
                sb1 = sb1 + rn1b[g, pl.ds(4, 16)]
            hs0 = jnp.maximum(rs[e, pl.ds(0, 16)] + sixth * sb0 + b1A, 0.0)
            hs1 = jnp.maximum(rs[e, pl.ds(4, 16)] + sixth * sb1 + b1B, 0.0)
            hb[e, pl.ds(0, 16)] = hs0
            hb[e, pl.ds(4, 16)] = hs1
            hb[e, pl.ds(20, 16)] = sixth * acc0
            hb[e, pl.ds(24, 16)] = sixth * acc1

        pltpu.sync_copy(hb, out_hbm.at[pl.ds(cb, CH)])


def _gather_kernel(F1, F2, idx, n1m, n2m, b1cat):
    mesh = plsc.VectorSubcoreMesh(core_axis_name="c", subcore_axis_name="s")
    run = functools.partial(
        pl.kernel,
        out_type=jax.ShapeDtypeStruct((BATCH, HD), jnp.float32),
        mesh=mesh,
        compiler_params=pltpu.CompilerParams(use_tc_tiling_on_sc=False),
        scratch_types=[
            pltpu.VMEM((CH,), jnp.int32),
            pltpu.VMEM((CH * FANOUT,), jnp.int32),
            pltpu.VMEM((CH * FANOUT * FANOUT,), jnp.int32),
            pltpu.VMEM((CH, TD), jnp.float32),
            pltpu.VMEM((CH * FANOUT, TD), jnp.float32),
            pltpu.VMEM((CH * FANOUT, TD), jnp.float32),
            pltpu.VMEM((CH * FANOUT * FANOUT, TD), jnp.float32),
            pltpu.VMEM((CH, HD), jnp.float32),
            pltpu.VMEM((32,), jnp.float32),
            pltpu.SemaphoreType.DMA,
        ],
    )(_gather_body)
    return run(F1, F2, idx, n1m, n2m, b1cat)


def kernel(features, idx, first_order_neighs, second_order_neighs,
           W1, b1, W2, b2):
    F1, F2 = _make_tables(features, W1)
    n1m = first_order_neighs.reshape(-1)
    n2m = second_order_neighs.reshape(-1)
    b1cat = jnp.concatenate([b1[0:16], b1[4:20]])
    H = _gather_kernel(F1, F2, idx, n1m, n2m, b1cat)
    return _head(H, W2, b2)


# speedup vs baseline: 7.7798x; 1.0789x over previous
import functools

import jax
import jax.numpy as jnp
from jax import lax
from jax.experimental import pallas as pl
from jax.experimental.pallas import tpu as pltpu
from jax.experimental.pallas import tpu_sc as plsc

N_NODES = 100000
IN_DIM = 60
BATCH = 16384
FANOUT = 6
DIMS = 20
TD = 24
HD = 128

NC, NS = 2, 16
NW = NC * NS
BPW = BATCH // NW
CH = 64
NCHUNK = BPW // CH
IDX_TILE = 128


def _table_body(x_ref, w1a_ref, w1b_ref, f1_ref, f2_ref):
    x = x_ref[...]
    f1_ref[...] = jnp.dot(x, w1a_ref[...], preferred_element_type=jnp.float32)
    f2_ref[...] = jnp.dot(x, w1b_ref[...], preferred_element_type=jnp.float32)


def _make_tables(features, W1):
    rows = features.shape[0]
    blk = 8192
    grid = (rows + blk - 1) // blk
    return pl.pallas_call(
        _table_body,
        grid=(grid,),
        in_specs=[
            pl.BlockSpec((blk, IN_DIM), lambda i: (i, 0)),
            pl.BlockSpec((IN_DIM, TD), lambda i: (0, 0)),
            pl.BlockSpec((IN_DIM, TD), lambda i: (0, 0)),
        ],
        out_specs=[
            pl.BlockSpec((blk, TD), lambda i: (i, 0)),
            pl.BlockSpec((blk, TD), lambda i: (i, 0)),
        ],
        out_shape=[
            jax.ShapeDtypeStruct((rows, TD), jnp.float32),
            jax.ShapeDtypeStruct((rows, TD), jnp.float32),
        ],
    )(features,
      jnp.pad(W1[:IN_DIM], ((0, 0), (0, TD - DIMS))),
      jnp.pad(W1[IN_DIM:], ((0, 0), (0, TD - DIMS))))


def _head_body(h_ref, w2_ref, b2_ref, o_ref):
    h = h_ref[...][:, :2 * DIMS]
    acc = jnp.dot(h, w2_ref[...], preferred_element_type=jnp.float32)
    o_ref[...] = jnp.maximum(acc + b2_ref[...], 0.0)


def _head(H, W2, b2):
    blk = 2048
    return pl.pallas_call(
        _head_body,
        grid=(BATCH // blk,),
        in_specs=[
            pl.BlockSpec((blk, HD), lambda i: (i, 0)),

            pl.BlockSpec((2 * DIMS, DIMS), lambda i: (0, 0)),
            pl.BlockSpec((1, DIMS), lambda i: (0, 0)),
        ],
        out_specs=pl.BlockSpec((blk, DIMS), lambda i: (i, 0)),
        out_shape=jax.ShapeDtypeStruct((BATCH, DIMS), jnp.float32),
    )(H, W2, b2.reshape(1, DIMS))


N1_ROWS = BPW * FANOUT // IDX_TILE
N2_ROWS = BPW * FANOUT * FANOUT // IDX_TILE


def _gather_body(f1_hbm, f2_hbm, idx_hbm, n1_hbm, n2_hbm, b1_hbm, out_hbm,
                 idxv, n1v, n2v, rs, rn1a, rn1b, rn2, hb, b1v, sem):
    wid = lax.axis_index("s") * NC + lax.axis_index("c")
    base = wid * BPW
    pltpu.sync_copy(b1_hbm, b1v)
    pltpu.sync_copy(idx_hbm.at[pl.ds(base, BPW)], idxv)
    pltpu.sync_copy(n1_hbm.at[pl.ds(wid * N1_ROWS, N1_ROWS)], n1v)
    pltpu.sync_copy(n2_hbm.at[pl.ds(wid * N2_ROWS, N2_ROWS)], n2v)
    b1A = b1v[pl.ds(0, 16)]
    b1B = b1v[pl.ds(16, 16)]
    sixth = jnp.float32(1.0 / FANOUT)

    @pl.loop(0, NCHUNK)
    def chunk(ci):
        cb = base + ci * CH
        cps = [pltpu.async_copy(f1_hbm.at[idxv.at[pl.ds(ci * CH, CH)]], rs, sem)]
        for k in range(CH * FANOUT // IDX_TILE):
            src = ci * (CH * FANOUT // IDX_TILE) + k
            dst = pl.ds(k * IDX_TILE, IDX_TILE)
            cps.append(pltpu.async_copy(f1_hbm.at[n1v.at[src]], rn1a.at[dst], sem))
            cps.append(pltpu.async_copy(f2_hbm.at[n1v.at[src]], rn1b.at[dst], sem))
        for k in range(CH * FANOUT * FANOUT // IDX_TILE):
            src = ci * (CH * FANOUT * FANOUT // IDX_TILE) + k
            dst = pl.ds(k * IDX_TILE, IDX_TILE)
            cps.append(pltpu.async_copy(f2_hbm.at[n2v.at[src]], rn2.at[dst], sem))
        for cp in cps:
            cp.wait()

        @pl.loop(0, CH)
        def elem(e):
            zero = jnp.zeros((16,), jnp.float32)
            acc0 = zero
            acc1 = zero
            sb0 = zero
            sb1 = zero
            for i in range(FANOUT):
                g = e * FANOUT + i
                s0 = zero
                s1 = zero
                for j in range(FANOUT):
                    r = g * FANOUT + j
                    s0 = s0 + rn2[r, pl.ds(0, 16)]
                    s1 = s1 + rn2[r, pl.ds(4, 16)]
                q0 = jnp.maximum(rn1a[g, pl.ds(0, 16)] + sixth * s0 + b1A, 0.0)
                q1 = jnp.maximum(rn1a[g, pl.ds(4, 16)] + sixth * s1 + b1B, 0.0)
                acc0 = acc0 + q0
                acc1 = acc1 + q1
                sb0 = sb0 + rn1b[g, pl.ds(0, 16)]
                sb1 = sb1 + rn1b[g, pl.ds(4, 16)]
            hs0 = jnp.maximum(rs[e, pl.ds(0, 16)] + sixth * sb0 + b1A, 0.0)
            hs1 = jnp.maximum(rs[e, pl.ds(4, 16)] + sixth * sb1 + b1B, 0.0)
            hb[e, pl.ds(0, 16)] = hs0
            hb[e, pl.ds(4, 16)] = hs1
            hb[e, pl.ds(20, 16)] = sixth * acc0
            hb[e, pl.ds(24, 16)] = sixth * acc1

        pltpu.sync_copy(hb, out_hbm.at[pl.ds(cb, CH)])


def _gather_kernel(F1, F2, idx, n1m, n2m, b1cat):
    mesh = plsc.VectorSubcoreMesh(core_axis_name="c", subcore_axis_name="s")
    run = functools.partial(
        pl.kernel,
        out_type=jax.ShapeDtypeStruct((BATCH, HD), jnp.float32),
        mesh=mesh,
        compiler_params=pltpu.CompilerParams(use_tc_tiling_on_sc=False),
        scratch_types=[
            pltpu.VMEM((BPW,), jnp.int32),
            pltpu.VMEM((N1_ROWS, IDX_TILE), jnp.int32),
            pltpu.VMEM((N2_ROWS, IDX_TILE), jnp.int32),
            pltpu.VMEM((CH, TD), jnp.float32),
            pltpu.VMEM((CH * FANOUT, TD), jnp.float32),
            pltpu.VMEM((CH * FANOUT, TD), jnp.float32),
            pltpu.VMEM((CH * FANOUT * FANOUT, TD), jnp.float32),
            pltpu.VMEM((CH, HD), jnp.float32),
            pltpu.VMEM((32,), jnp.float32),
            pltpu.SemaphoreType.DMA,
        ],
    )(_gather_body)
    return run(F1, F2, idx, n1m, n2m, b1cat)


def kernel(features, idx, first_order_neighs, second_order_neighs,
           W1, b1, W2, b2):
    F1, F2 = _make_tables(features, W1)
    n1m = first_order_neighs.reshape(-1, IDX_TILE)
    n2m = second_order_neighs.reshape(-1, IDX_TILE)
    b1cat = jnp.concatenate([b1[0:16], b1[4:20]])
    H = _gather_kernel(F1, F2, idx, n1m, n2m, b1cat)
    return _head(H, W2, b2)
